# Initial kernel scaffold; baseline (speedup 1.0000x reference)
import jax
import jax.numpy as jnp
from jax.experimental import pallas as pl


def kernel(user, item, attr, edge_index, ent_emb, W1, b1, W2, b2, W3, b3):
    raise NotImplementedError("write your pallas kernel here")


# speedup vs baseline: 11.4244x; 11.4244x over previous
import functools

import jax
import jax.numpy as jnp
from jax import lax
from jax.experimental import pallas as pl
from jax.experimental.pallas import tpu as pltpu
from jax.experimental.pallas import tpu_sc as plsc

N_USER = 50000
N_ITEM = 50000
N_ATTR = 1000
N_ENT = N_USER + N_ITEM + N_ATTR + 1
D = 16
E = 3200000
B = 4096

NC = 2
NS = 16
NEPAD = 101120
ROWS_TILE = NEPAD // NS
K = 80
EP_TILE = E // (NC * NS)
NCHUNK = EP_TILE // K
FLAT_ROWS = NEPAD * D // 128
INV_ROWS = NEPAD // 128

_mesh = functools.partial(
    plsc.VectorSubcoreMesh, core_axis_name="c", subcore_axis_name="s")
_SC_PARAMS = pltpu.CompilerParams(
    needs_layout_passes=False, use_tc_tiling_on_sc=False)


def _iota16():
    return lax.iota(jnp.int32, 16)


def _zero_vmem_2d(ref, rows):
    z = jnp.zeros((16,), jnp.float32)
    for i in range(rows):
        ref[i, :] = z


def _zero_vmem_1d(ref, n):
    z = jnp.zeros((16,), jnp.float32)
    for g in range(n // 16):
        ref[pl.ds(g * 16, 16)] = z


def _edge_softmax_body(emb, src, dst, ex_out, dpart,
                       sidx, didx, srows, drows, exbuf, zbuf, dspm,
                       sem1, sem2):
    c = lax.axis_index("c")
    s = lax.axis_index("s")
    ebase = c * (NS * EP_TILE) + s * EP_TILE

    _zero_vmem_1d(zbuf, K)
    rbase = s * ROWS_TILE

    def zinit(i, _):
        pltpu.sync_copy(zbuf, dspm.at[pl.ds(rbase + i * K, K)])
        return 0

    lax.fori_loop(0, ROWS_TILE // K, zinit, 0)
    plsc.subcore_barrier()

    def step(ci, _):
        off = ebase + ci * K
        pltpu.sync_copy(src.at[pl.ds(off, K)], sidx)
        pltpu.sync_copy(dst.at[pl.ds(off, K)], didx)
        d1 = pltpu.async_copy(emb.at[sidx], srows, sem1)
        d2 = pltpu.async_copy(emb.at[didx], drows, sem2)
        d1.wait()
        d2.wait()
        for g in range(K // 16):
            row16 = g * 16 + _iota16()
            acc = jnp.zeros((16,), jnp.float32)
            for j in range(D):
                cj = jnp.full((16,), j, jnp.int32)
                a = plsc.load_gather(srows, [row16, cj])
                b = plsc.load_gather(drows, [row16, cj])
                acc = acc + a * b
            exbuf[pl.ds(g * 16, 16)] = jnp.exp(acc)
        pltpu.sync_copy(exbuf, ex_out.at[pl.ds(off, K)])
        pltpu.sync_copy(exbuf, dspm.at[didx], add=True)
        return 0

    lax.fori_loop(0, NCHUNK, step, 0)
    plsc.subcore_barrier()
    pltpu.sync_copy(dspm.at[pl.ds(rbase, ROWS_TILE)],
                    dpart.at[pl.ds(c * NEPAD + rbase, ROWS_TILE)])


_edge_softmax = pl.kernel(
    _edge_softmax_body,
    out_type=(jax.ShapeDtypeStruct((E,), jnp.float32),
              jax.ShapeDtypeStruct((NC * NEPAD,), jnp.float32)),
    mesh=_mesh(),
    compiler_params=_SC_PARAMS,
    scratch_types=[
        pltpu.VMEM((K,), jnp.int32),
        pltpu.VMEM((K,), jnp.int32),
        pltpu.VMEM((K, D), jnp.float32),
        pltpu.VMEM((K, D), jnp.float32),
        pltpu.VMEM((K,), jnp.float32),
        pltpu.VMEM((K,), jnp.float32),
        pltpu.VMEM_SHARED((NEPAD,), jnp.float32),
        pltpu.SemaphoreType.DMA,
        pltpu.SemaphoreType.DMA,
    ],
)


def _agg1_body(htab, src, dst, ex, inv, w_out, part,
               sidx, didx, exv, wv, ivals, rows, zbuf, invspm, accspm,
               sem1, sem2):
    c = lax.axis_index("c")
    s = lax.axis_index("s")
    ebase = c * (NS * EP_TILE) + s * EP_TILE

    _zero_vmem_2d(zbuf, K)
    rbase = s * ROWS_TILE
    pltpu.sync_copy(inv.at[pl.ds(rbase, ROWS_TILE)],
                    invspm.at[pl.ds(rbase, ROWS_TILE)])

    def zinit(i, _):
        pltpu.sync_copy(zbuf, accspm.at[pl.ds(rbase + i * K, K), :])
        return 0

    lax.fori_loop(0, ROWS_TILE // K, zinit, 0)
    plsc.subcore_barrier()

    def step(ci, _):
        off = ebase + ci * K
        pltpu.sync_copy(src.at[pl.ds(off, K)], sidx)
        pltpu.sync_copy(dst.at[pl.ds(off, K)], didx)
        pltpu.sync_copy(ex.at[pl.ds(off, K)], exv)
        d1 = pltpu.async_copy(htab.at[sidx], rows, sem1)
        d2 = pltpu.async_copy(invspm.at[didx], ivals, sem2)
        d1.wait()
        d2.wait()
        for g in range(K // 16):
            w16 = (exv[pl.ds(g * 16, 16)] * ivals[pl.ds(g * 16, 16)])
            wv[pl.ds(g * 16, 16)] = w16
            row16 = g * 16 + _iota16()
            for j in range(D):
                cj = jnp.full((16,), j, jnp.int32)
                v = plsc.load_gather(rows, [row16, cj])
                plsc.store_scatter(rows, [row16, cj], v * w16)
        pltpu.sync_copy(wv, w_out.at[pl.ds(off, K)])
        pltpu.sync_copy(rows, accspm.at[didx], add=True)
        return 0

    lax.fori_loop(0, NCHUNK, step, 0)
    plsc.subcore_barrier()
    pltpu.sync_copy(accspm.at[pl.ds(rbase, ROWS_TILE), :],
                    part.at[c, pl.ds(rbase, ROWS_TILE), :])


_agg1 = pl.kernel(
    _agg1_body,
    out_type=(jax.ShapeDtypeStruct((E,), jnp.float32),
              jax.ShapeDtypeStruct((NC, NEPAD, D), jnp.float32)),
    mesh=_mesh(),
    compiler_params=_SC_PARAMS,
    scratch_types=[
        pltpu.VMEM((K,), jnp.int32),
        pltpu.VMEM((K,), jnp.int32),
        pltpu.VMEM((K,), jnp.float32),
        pltpu.VMEM((K,), jnp.float32),
        pltpu.VMEM((K,), jnp.float32),
        pltpu.VMEM((K, D), jnp.float32),
        pltpu.VMEM((K, D), jnp.float32),
        pltpu.VMEM_SHARED((NEPAD,), jnp.float32),
        pltpu.VMEM_SHARED((NEPAD, D), jnp.float32),
        pltpu.SemaphoreType.DMA,
        pltpu.SemaphoreType.DMA,
    ],
)


def _agg2_body(htab, src, dst, w, part,
               sidx, didx, wv, rows, zbuf, accspm, sem1):
    c = lax.axis_index("c")
    s = lax.axis_index("s")
    ebase = c * (NS * EP_TILE) + s * EP_TILE

    _zero_vmem_2d(zbuf, K)
    rbase = s * ROWS_TILE

    def zinit(i, _):
        pltpu.sync_copy(zbuf, accspm.at[pl.ds(rbase + i * K, K), :])
        return 0

    lax.fori_loop(0, ROWS_TILE // K, zinit, 0)
    plsc.subcore_barrier()

    def step(ci, _):
        off = ebase + ci * K
        pltpu.sync_copy(src.at[pl.ds(off, K)], sidx)
        pltpu.sync_copy(dst.at[pl.ds(off, K)], didx)
        pltpu.sync_copy(w.at[pl.ds(off, K)], wv)
        pltpu.async_copy(htab.at[sidx], rows, sem1).wait()
        for g in range(K // 16):
            w16 = wv[pl.ds(g * 16, 16)]
            row16 = g * 16 + _iota16()
            for j in range(D):
                cj = jnp.full((16,), j, jnp.int32)
                v = plsc.load_gather(rows, [row16, cj])
                plsc.store_scatter(rows, [row16, cj], v * w16)
        pltpu.sync_copy(rows, accspm.at[didx], add=True)
        return 0

    lax.fori_loop(0, NCHUNK, step, 0)
    plsc.subcore_barrier()
    pltpu.sync_copy(accspm.at[pl.ds(rbase, ROWS_TILE), :],
                    part.at[c, pl.ds(rbase, ROWS_TILE), :])


_agg2 = pl.kernel(
    _agg2_body,
    out_type=jax.ShapeDtypeStruct((NC, NEPAD, D), jnp.float32),
    mesh=_mesh(),
    compiler_params=_SC_PARAMS,
    scratch_types=[
        pltpu.VMEM((K,), jnp.int32),
        pltpu.VMEM((K,), jnp.int32),
        pltpu.VMEM((K,), jnp.float32),
        pltpu.VMEM((K, D), jnp.float32),
        pltpu.VMEM((K, D), jnp.float32),
        pltpu.VMEM_SHARED((NEPAD, D), jnp.float32),
        pltpu.SemaphoreType.DMA,
    ],
)


BT = B // (NC * NS)


def _batch_gather_body(final, user, item, attr, out, idxv, rowsv, sem):
    c = lax.axis_index("c")
    s = lax.axis_index("s")
    tbase = (c * NS + s) * BT
    ids = (user, item, attr)
    offs = (0, N_USER, N_USER + N_ITEM)
    for sec in range(3):
        pltpu.sync_copy(ids[sec].at[pl.ds(tbase, BT)], idxv)
        if offs[sec]:
            off = jnp.full((16,), offs[sec], jnp.int32)
            for g in range(BT // 16):
                idxv[pl.ds(g * 16, 16)] = idxv[pl.ds(g * 16, 16)] + off
        pltpu.async_copy(final.at[idxv], rowsv, sem).wait()
        pltpu.sync_copy(rowsv, out.at[sec, pl.ds(tbase, BT), :])


_batch_gather = pl.kernel(
    _batch_gather_body,
    out_type=jax.ShapeDtypeStruct((3, B, D), jnp.float32),
    mesh=_mesh(),
    compiler_params=_SC_PARAMS,
    scratch_types=[
        pltpu.VMEM((BT,), jnp.int32),
        pltpu.VMEM((BT, D), jnp.float32),
        pltpu.SemaphoreType.DMA,
    ],
)


def _inv_body(d0, d1, o):
    o[...] = 1.0 / (d0[...] + d1[...] + 1e-9)


def _tc_inv(d0, d1):
    return pl.pallas_call(
        _inv_body,
        out_shape=jax.ShapeDtypeStruct((INV_ROWS, 128), jnp.float32),
    )(d0, d1)


def _fin_body(scale, p0, p1, acc, h, accn):
    hp = jnp.tanh(p0[...] + p1[...])
    h[...] = hp
    accn[...] = (acc[...] + hp) * scale


def _tc_finalize(p0, p1, acc, scale):
    blk = FLAT_ROWS // 10
    body = functools.partial(_fin_body, scale)
    return pl.pallas_call(
        body,
        grid=(10,),
        in_specs=[pl.BlockSpec((blk, 128), lambda i: (i, 0))] * 3,
        out_specs=[pl.BlockSpec((blk, 128), lambda i: (i, 0))] * 2,
        out_shape=(jax.ShapeDtypeStruct((FLAT_ROWS, 128), jnp.float32),
                   jax.ShapeDtypeStruct((FLAT_ROWS, 128), jnp.float32)),
    )(p0, p1, acc)


def _mlp_body(ue, ie, ae, w1, b1, w2, b2, w3, b3, o):
    x = (jnp.dot(ue[...], w1[0:D, :], preferred_element_type=jnp.float32)
         + jnp.dot(ie[...], w1[D:2 * D, :], preferred_element_type=jnp.float32)
         + jnp.dot(ae[...], w1[2 * D:3 * D, :],
                   preferred_element_type=jnp.float32))
    h1 = jnp.tanh(x + b1[...])
    h2 = jnp.tanh(jnp.dot(h1, w2[...], preferred_element_type=jnp.float32)
                  + b2[...])
    o[...] = jnp.dot(h2, w3[...], preferred_element_type=jnp.float32) + b3[...]


def _tc_mlp(ue, ie, ae, w1, b1, w2, b2, w3, b3):
    rb = 512
    full = lambda shape: pl.BlockSpec(shape, lambda i: (0, 0))
    return pl.pallas_call(
        _mlp_body,
        grid=(B // rb,),
        in_specs=[
            pl.BlockSpec((rb, D), lambda i: (i, 0)),
            pl.BlockSpec((rb, D), lambda i: (i, 0)),
            pl.BlockSpec((rb, D), lambda i: (i, 0)),
            full((3 * D, 64)), full((1, 64)),
            full((64, 32)), full((1, 32)),
            full((32, 1)), full((1, 1)),
        ],
        out_specs=pl.BlockSpec((rb, 1), lambda i: (i, 0)),
        out_shape=jax.ShapeDtypeStruct((B, 1), jnp.float32),
    )(ue, ie, ae, w1, b1, w2, b2, w3, b3)


def kernel(user, item, attr, edge_index, ent_emb, W1, b1, W2, b2, W3, b3):
    src = edge_index[0]
    dst = edge_index[1]

    ex, dparts = _edge_softmax(ent_emb, src, dst)
    dparts = dparts.reshape(NC, INV_ROWS, 128)
    inv = _tc_inv(dparts[0], dparts[1]).reshape(NEPAD)

    acc0 = jnp.pad(ent_emb, ((0, NEPAD - N_ENT), (0, 0)))
    w, p = _agg1(ent_emb, src, dst, ex, inv)
    h1f, acc1 = _tc_finalize(p[0].reshape(FLAT_ROWS, 128),
                             p[1].reshape(FLAT_ROWS, 128),
                             acc0.reshape(FLAT_ROWS, 128), 1.0)
    h1 = h1f.reshape(NEPAD, D)

    q = _agg2(h1, src, dst, w)
    _, finalf = _tc_finalize(q[0].reshape(FLAT_ROWS, 128),
                             q[1].reshape(FLAT_ROWS, 128),
                             acc1, 1.0 / 3.0)
    final = finalf.reshape(NEPAD, D)

    g3 = _batch_gather(final, user, item, attr)
    return _tc_mlp(g3[0], g3[1], g3[2],
                   W1, b1.reshape(1, 64), W2, b2.reshape(1, 32),
                   W3, b3.reshape(1, 1))
